# baseline (speedup 1.0000x reference)
import functools

import jax
import jax.numpy as jnp
from jax import lax
from jax.experimental import pallas as pl
from jax.experimental.pallas import tpu as pltpu
from jax.experimental.pallas import tpu_sc as plsc

B = 16384
R = 8
V = 64
NDIMS = 7
NC, NS, L = 2, 16, 16
NW = NC * NS
BPW = B // NW
GROUPS = BPW // L
SE = R + 1
SM = R * R + 1
MIDSZ = V * SM


def _tt_body(idx_hbm, ends_hbm, mids_hbm, out_hbm,
             idx_v, ends_v, mids_v, out_v, sem):
    wid = lax.axis_index("s") * NC + lax.axis_index("c")
    base = wid * BPW

    copies = [
        pltpu.async_copy(ends_hbm, ends_v, sem),
        pltpu.async_copy(mids_hbm, mids_v, sem),
        pltpu.async_copy(idx_hbm.at[pl.ds(wid * (NDIMS * BPW), NDIMS * BPW)],
                         idx_v, sem),
    ]
    for c in copies:
        c.wait()

    @plsc.parallel_loop(0, GROUPS)
    def _group(g):
        o = g * L
        i0 = idx_v[pl.ds(0 * BPW + o, L)] * SE
        res = tuple(plsc.load_gather(ends_v, [i0 + j]) for j in range(R))

        def mid(k, res_c):
            ik = (idx_v[pl.ds(k * BPW + o, L)] * SM
                  + (k - 1) * MIDSZ)
            new = []
            for l in range(R):
                acc = res_c[0] * plsc.load_gather(mids_v, [ik + l])
                for j in range(1, R):
                    acc = acc + res_c[j] * plsc.load_gather(
                        mids_v, [ik + (j * R + l)])
                new.append(acc)
            return tuple(new)

        res = lax.fori_loop(1, 6, mid, res)

        i6 = idx_v[pl.ds(6 * BPW + o, L)] * SE + (V * SE)
        q = res[0] * plsc.load_gather(ends_v, [i6 + 0])
        for j in range(1, R):
            q = q + res[j] * plsc.load_gather(ends_v, [i6 + j])
        out_v[pl.ds(o, L)] = q

    pltpu.sync_copy(out_v, out_hbm.at[pl.ds(base, BPW)])


_tt_gather = functools.partial(
    pl.kernel,
    out_type=jax.ShapeDtypeStruct((B,), jnp.float32),
    mesh=plsc.VectorSubcoreMesh(core_axis_name="c", subcore_axis_name="s",
                                num_cores=NC, num_subcores=NS),
    compiler_params=pltpu.CompilerParams(needs_layout_passes=False),
    scratch_types=[
        pltpu.VMEM((NDIMS * BPW,), jnp.int32),
        pltpu.VMEM((2 * V * SE,), jnp.float32),
        pltpu.VMEM((5 * MIDSZ,), jnp.float32),
        pltpu.VMEM((BPW,), jnp.float32),
        pltpu.SemaphoreType.DMA,
    ],
)(_tt_body)


def kernel(states, actions, core0, core1, core2, core3, core4, core5, core6):
    idxp = (jnp.concatenate([states.T, actions.T], axis=0)
            .reshape(NDIMS, NW, BPW).transpose(1, 0, 2).reshape(-1))
    ends = jnp.concatenate(
        [core0.reshape(V, R), jnp.transpose(core6, (1, 0, 2)).reshape(V, R)],
        axis=0)
    endsp = jnp.pad(ends, ((0, 0), (0, SE - R))).reshape(-1)
    mids = jnp.stack([core1, core2, core3, core4, core5], axis=0)
    midsp = jnp.pad(mids.transpose(0, 2, 1, 3).reshape(5 * V, R * R),
                    ((0, 0), (0, SM - R * R))).reshape(-1)
    return _tt_gather(idxp, endsp, midsp)
